# baseline (speedup 1.0000x reference)
import jax
import jax.numpy as jnp
from jax import lax
from jax.experimental import pallas as pl
from jax.experimental.pallas import tpu as pltpu

N = 4096
C = 3
H = 256
W = 256

PIX_BLK = 1024
N_BLK = 512
N_ROWS = PIX_BLK // W


def _raster_body(xyz_ref, chol_ref, colors_ref, out_ref):
    p = pl.program_id(0)
    nb = pl.program_id(1)

    xy = jnp.tanh(xyz_ref[...])
    cx = 0.5 * W * (xy[0:1, :] + 1.0)
    cy = 0.5 * H * (xy[1:2, :] + 1.0)
    l11 = chol_ref[0:1, :]
    l21 = chol_ref[1:2, :]
    l22 = chol_ref[2:3, :]
    s11 = l11 * l11
    s12 = l11 * l21
    s22 = l21 * l21 + l22 * l22
    det = s11 * s22 - s12 * s12
    inv_det = 1.0 / det
    a = (0.5 * s22) * inv_det
    b = (-s12) * inv_det
    c = (0.5 * s11) * inv_det

    lin = lax.broadcasted_iota(jnp.int32, (PIX_BLK, 1), 0)
    row0 = (p * N_ROWS).astype(jnp.float32)
    xs = (lin % W).astype(jnp.float32) + 0.5
    ys = (lin // W).astype(jnp.float32) + (row0 + 0.5)

    dx = xs - cx
    dy = ys - cy
    sigma = (a * dx + b * dy) * dx + c * (dy * dy)
    wgt = jnp.exp(-jnp.maximum(sigma, 0.0))

    contrib = jnp.dot(wgt, colors_ref[...], preferred_element_type=jnp.float32)

    @pl.when(nb == 0)
    def _():
        out_ref[...] = contrib

    @pl.when(nb != 0)
    def _():
        out_ref[...] += contrib


def kernel(_xyz, _cholesky, _features_dc, _opacity, cholesky_bound, render_colors):
    colors = jnp.transpose(_features_dc, (1, 0, 2)).reshape(N, C * 3)
    colors = colors * _opacity[:, 0:1]
    colors = jnp.pad(colors, ((0, 0), (0, 16 - C * 3)))
    xyz_t = _xyz.T
    chol_t = (_cholesky + cholesky_bound).T

    grid = (H * W // PIX_BLK, N // N_BLK)
    out_flat = pl.pallas_call(
        _raster_body,
        grid=grid,
        in_specs=[
            pl.BlockSpec((2, N_BLK), lambda p, n: (0, n)),
            pl.BlockSpec((3, N_BLK), lambda p, n: (0, n)),
            pl.BlockSpec((N_BLK, 16), lambda p, n: (n, 0)),
        ],
        out_specs=pl.BlockSpec((PIX_BLK, 16), lambda p, n: (p, 0)),
        out_shape=jax.ShapeDtypeStruct((H * W, 16), jnp.float32),
        compiler_params=pltpu.CompilerParams(
            dimension_semantics=("parallel", "arbitrary"),
        ),
    )(xyz_t, chol_t, colors)

    out = out_flat[:, : C * 3].reshape(H, W, C, 3)
    return jnp.transpose(out, (2, 3, 0, 1))
